# baseline (speedup 1.0000x reference)
import functools
import math

import jax
import jax.numpy as jnp
from jax import lax
from jax.experimental import pallas as pl
from jax.experimental.pallas import tpu as pltpu
from jax.experimental.pallas import tpu_sc as plsc

_N = 4096
_VOCAB = 32000
_IGNORE = 0
_EPS = 0.1
_U = _EPS / (_VOCAB - 2)
_C = (1.0 - _EPS) * math.log(1.0 - _EPS) + (_VOCAB - 2) * _U * math.log(_U)

_ROW_BLOCK = 128

_NUM_WORKERS = 32
_ROWS_PER_W = _N // _NUM_WORKERS
_LANES = 16


def _sc_gather_body(lp_flat_hbm, tgt_hbm, a_out, m_out,
                    tgt_v, idx_v, idx0_v, lpt_v, lp0_v, a_v, m_v, sem):
    wid = lax.axis_index("s") * 2 + lax.axis_index("c")
    base = wid * _ROWS_PER_W
    pltpu.sync_copy(tgt_hbm.at[pl.ds(base, _ROWS_PER_W)], tgt_v)
    for k in range(_ROWS_PER_W // _LANES):
        rows = base + k * _LANES + lax.iota(jnp.int32, _LANES)
        tvec = tgt_v[pl.ds(k * _LANES, _LANES)]
        idx_v[pl.ds(k * _LANES, _LANES)] = rows * _VOCAB + tvec
        idx0_v[pl.ds(k * _LANES, _LANES)] = rows * _VOCAB
    pltpu.async_copy(lp_flat_hbm.at[idx_v], lpt_v, sem).wait()
    pltpu.async_copy(lp_flat_hbm.at[idx0_v], lp0_v, sem).wait()
    for k in range(_ROWS_PER_W // _LANES):
        sl = pl.ds(k * _LANES, _LANES)
        tvec = tgt_v[sl]
        nonpad = tvec != _IGNORE
        a = _C + _U * lp0_v[sl] - (1.0 - _EPS - _U) * lpt_v[sl]
        a_v[sl] = jnp.where(nonpad, a, 0.0)
        m_v[sl] = jnp.where(nonpad, jnp.float32(_U), 0.0)
    pltpu.sync_copy(a_v, a_out.at[pl.ds(base, _ROWS_PER_W)])
    pltpu.sync_copy(m_v, m_out.at[pl.ds(base, _ROWS_PER_W)])


def _sc_gather(log_probs, targets):
    lp_flat = log_probs.reshape(-1)
    mesh = plsc.VectorSubcoreMesh(core_axis_name="c", subcore_axis_name="s")
    run = pl.kernel(
        _sc_gather_body, mesh=mesh,
        out_type=[jax.ShapeDtypeStruct((_N,), jnp.float32),
                  jax.ShapeDtypeStruct((_N,), jnp.float32)],
        scratch_types=[
            pltpu.VMEM((_ROWS_PER_W,), jnp.int32),
            pltpu.VMEM((_ROWS_PER_W,), jnp.int32),
            pltpu.VMEM((_ROWS_PER_W,), jnp.int32),
            pltpu.VMEM((_ROWS_PER_W,), jnp.float32),
            pltpu.VMEM((_ROWS_PER_W,), jnp.float32),
            pltpu.VMEM((_ROWS_PER_W,), jnp.float32),
            pltpu.VMEM((_ROWS_PER_W,), jnp.float32),
            pltpu.SemaphoreType.DMA,
        ],
    )
    return run(lp_flat, targets)


def _tc_loss_kernel(a_ref, m_ref, lp_ref, out_ref):
    i = pl.program_id(0)
    tile = lp_ref[...]
    rowsum = jnp.sum(tile, axis=1)
    a = a_ref[0, 0, :]
    m = m_ref[0, 0, :]
    contrib = a - m * rowsum
    partial = jnp.sum(contrib.reshape(1, -1), axis=1, keepdims=True)

    @pl.when(i == 0)
    def _():
        out_ref[...] = jnp.zeros_like(out_ref)

    out_ref[...] += partial


def kernel(log_probs, targets):
    n, v = log_probs.shape
    r = _ROW_BLOCK
    nb = n // r
    a, m = _sc_gather(log_probs, targets)
    out = pl.pallas_call(
        _tc_loss_kernel,
        grid=(nb,),
        in_specs=[
            pl.BlockSpec((1, 1, r), lambda i: (i, 0, 0)),
            pl.BlockSpec((1, 1, r), lambda i: (i, 0, 0)),
            pl.BlockSpec((r, v), lambda i: (i, 0)),
        ],
        out_specs=pl.BlockSpec((1, 1), lambda i: (0, 0)),
        out_shape=jax.ShapeDtypeStruct((1, 1), jnp.float32),
    )(a.reshape(nb, 1, r), m.reshape(nb, 1, r), log_probs)
    return out[0, 0]


# speedup vs baseline: 2.9601x; 2.9601x over previous
import math

import jax
import jax.numpy as jnp
from jax import lax
from jax.experimental import pallas as pl
from jax.experimental.pallas import tpu as pltpu
from jax.experimental.pallas import tpu_sc as plsc

_N = 4096
_VOCAB = 32000
_IGNORE = 0
_EPS = 0.1
_U = _EPS / (_VOCAB - 2)
_C = (1.0 - _EPS) * math.log(1.0 - _EPS) + (_VOCAB - 2) * _U * math.log(_U)

_ROW_BLOCK = 128
_LANES = 16


def _tc_stream_kernel(tgt_ref, lp_ref, rs_ref, lpt_ref, lp0_ref):
    tile = lp_ref[...]
    tgt = tgt_ref[0, 0, :]
    cols = jax.lax.broadcasted_iota(jnp.int32, tile.shape, 1)
    rs_ref[0, 0, :] = jnp.sum(tile, axis=1)
    lpt_ref[0, 0, :] = jnp.sum(jnp.where(cols == tgt[:, None], tile, 0.0), axis=1)
    lp0_ref[0, 0, :] = tile[:, 0]


def _sc_combine_body(tgt_hbm, rs_hbm, lpt_hbm, lp0_hbm, out_hbm,
                     tgt_v, rs_v, lpt_v, lp0_v, red_v, out_v):
    wid = lax.axis_index("s") * 2 + lax.axis_index("c")
    pltpu.sync_copy(tgt_hbm, tgt_v)
    pltpu.sync_copy(rs_hbm, rs_v)
    pltpu.sync_copy(lpt_hbm, lpt_v)
    pltpu.sync_copy(lp0_hbm, lp0_v)

    def body(k, acc):
        sl = pl.ds(k * _LANES, _LANES)
        tgt = tgt_v[sl]
        loss = (_C + _U * lp0_v[sl]
                - (1.0 - _EPS - _U) * lpt_v[sl]
                - _U * rs_v[sl])
        return acc + jnp.where(tgt == _IGNORE, 0.0, loss)

    acc = lax.fori_loop(0, _N // _LANES, body,
                        jnp.zeros((_LANES,), jnp.float32))
    red_v[pl.ds(_LANES, _LANES)] = jnp.zeros((_LANES,), jnp.float32)
    for shift in (8, 4, 2, 1):
        red_v[pl.ds(0, _LANES)] = acc
        acc = acc + red_v[pl.ds(shift, _LANES)]
    out_v[...] = acc

    @pl.when(wid == 0)
    def _():
        pltpu.sync_copy(out_v, out_hbm)


def kernel(log_probs, targets):
    n, v = log_probs.shape
    r = _ROW_BLOCK
    nb = n // r
    blk = jax.ShapeDtypeStruct((nb, 1, r), jnp.float32)
    rs, lpt, lp0 = pl.pallas_call(
        _tc_stream_kernel,
        grid=(nb,),
        in_specs=[
            pl.BlockSpec((1, 1, r), lambda i: (i, 0, 0)),
            pl.BlockSpec((r, v), lambda i: (i, 0)),
        ],
        out_specs=[pl.BlockSpec((1, 1, r), lambda i: (i, 0, 0))] * 3,
        out_shape=[blk, blk, blk],
    )(targets.reshape(nb, 1, r), log_probs)

    mesh = plsc.VectorSubcoreMesh(core_axis_name="c", subcore_axis_name="s")
    combine = pl.kernel(
        _sc_combine_body, mesh=mesh,
        out_type=jax.ShapeDtypeStruct((_LANES,), jnp.float32),
        scratch_types=[
            pltpu.VMEM((_N,), jnp.int32),
            pltpu.VMEM((_N,), jnp.float32),
            pltpu.VMEM((_N,), jnp.float32),
            pltpu.VMEM((_N,), jnp.float32),
            pltpu.VMEM((2 * _LANES,), jnp.float32),
            pltpu.VMEM((_LANES,), jnp.float32),
        ],
    )
    loss16 = combine(targets, rs.reshape(n), lpt.reshape(n), lp0.reshape(n))
    return loss16[0]


# speedup vs baseline: 2.9613x; 1.0004x over previous
import math

import jax
import jax.numpy as jnp
from jax import lax
from jax.experimental import pallas as pl
from jax.experimental.pallas import tpu as pltpu
from jax.experimental.pallas import tpu_sc as plsc

_N = 4096
_VOCAB = 32000
_IGNORE = 0
_EPS = 0.1
_U = _EPS / (_VOCAB - 2)
_C = (1.0 - _EPS) * math.log(1.0 - _EPS) + (_VOCAB - 2) * _U * math.log(_U)

_ROW_BLOCK = 128
_LANES = 16


def _tc_stream_kernel(tgt_ref, lp_ref, contrib_ref):
    tile = lp_ref[...]
    tgt = tgt_ref[0, 0, :]
    cols = jax.lax.broadcasted_iota(jnp.int32, tile.shape, 1)
    rowsum = jnp.sum(tile, axis=1)
    lp_t = jnp.sum(jnp.where(cols == tgt[:, None], tile, 0.0), axis=1)
    lp_0 = tile[:, 0]
    loss = _C + _U * lp_0 - (1.0 - _EPS - _U) * lp_t - _U * rowsum
    contrib_ref[0, 0, :] = jnp.where(tgt == _IGNORE, 0.0, loss)


def _sc_reduce_body(contrib_hbm, out_hbm, contrib_v, red_v, out_v):
    wid = lax.axis_index("s") * 2 + lax.axis_index("c")

    @pl.when(wid == 0)
    def _():
        pltpu.sync_copy(contrib_hbm, contrib_v)

        def body(k, acc):
            return acc + contrib_v[pl.ds(k * _LANES, _LANES)]

        acc = lax.fori_loop(0, _N // _LANES, body,
                            jnp.zeros((_LANES,), jnp.float32))
        red_v[pl.ds(_LANES, _LANES)] = jnp.zeros((_LANES,), jnp.float32)
        for shift in (8, 4, 2, 1):
            red_v[pl.ds(0, _LANES)] = acc
            acc = acc + red_v[pl.ds(shift, _LANES)]
        out_v[...] = acc
        pltpu.sync_copy(out_v, out_hbm)


def kernel(log_probs, targets):
    n, v = log_probs.shape
    r = _ROW_BLOCK
    nb = n // r
    contrib = pl.pallas_call(
        _tc_stream_kernel,
        grid=(nb,),
        in_specs=[
            pl.BlockSpec((1, 1, r), lambda i: (i, 0, 0)),
            pl.BlockSpec((r, v), lambda i: (i, 0)),
        ],
        out_specs=pl.BlockSpec((1, 1, r), lambda i: (i, 0, 0)),
        out_shape=jax.ShapeDtypeStruct((nb, 1, r), jnp.float32),
    )(targets.reshape(nb, 1, r), log_probs)

    mesh = plsc.VectorSubcoreMesh(core_axis_name="c", subcore_axis_name="s")
    reduce = pl.kernel(
        _sc_reduce_body, mesh=mesh,
        out_type=jax.ShapeDtypeStruct((_LANES,), jnp.float32),
        scratch_types=[
            pltpu.VMEM((_N,), jnp.float32),
            pltpu.VMEM((2 * _LANES,), jnp.float32),
            pltpu.VMEM((_LANES,), jnp.float32),
        ],
    )
    loss16 = reduce(contrib.reshape(n))
    return loss16[0]


# speedup vs baseline: 3.0631x; 1.0344x over previous
import math

import jax
import jax.numpy as jnp
from jax import lax
from jax.experimental import pallas as pl
from jax.experimental.pallas import tpu as pltpu
from jax.experimental.pallas import tpu_sc as plsc

_N = 4096
_VOCAB = 32000
_IGNORE = 0
_EPS = 0.1
_U = _EPS / (_VOCAB - 2)
_C = (1.0 - _EPS) * math.log(1.0 - _EPS) + (_VOCAB - 2) * _U * math.log(_U)

_ROW_BLOCK = 128
_LANES = 16


def _tc_stream_kernel(tgt_ref, lp_ref, contrib_ref):
    tile = lp_ref[...]
    tgt = tgt_ref[0, 0, :]
    cols = jax.lax.broadcasted_iota(jnp.int32, tile.shape, 1)
    rowsum = jnp.sum(tile, axis=1)
    lp_t = jnp.sum(jnp.where(cols == tgt[:, None], tile, 0.0), axis=1)
    lp_0 = tile[:, 0]
    loss = _C + _U * lp_0 - (1.0 - _EPS - _U) * lp_t - _U * rowsum
    contrib_ref[0, 0, :] = jnp.where(tgt == _IGNORE, 0.0, loss)


def _sc_reduce_body(contrib_hbm, out_hbm, contrib_v, red_v, out_v):
    wid = lax.axis_index("s") * 2 + lax.axis_index("c")

    @pl.when(wid == 0)
    def _():
        pltpu.sync_copy(contrib_hbm, contrib_v)

        def body(k, acc):
            return acc + contrib_v[pl.ds(k * _LANES, _LANES)]

        acc = lax.fori_loop(0, _N // _LANES, body,
                            jnp.zeros((_LANES,), jnp.float32))
        red_v[pl.ds(_LANES, _LANES)] = jnp.zeros((_LANES,), jnp.float32)
        for shift in (8, 4, 2, 1):
            red_v[pl.ds(0, _LANES)] = acc
            acc = acc + red_v[pl.ds(shift, _LANES)]
        out_v[...] = acc
        pltpu.sync_copy(out_v, out_hbm)


def kernel(log_probs, targets):
    n, v = log_probs.shape
    r = _ROW_BLOCK
    nb = n // r
    contrib = pl.pallas_call(
        _tc_stream_kernel,
        grid=(nb,),
        in_specs=[
            pl.BlockSpec((1, 1, r), lambda i: (i, 0, 0)),
            pl.BlockSpec((r, v), lambda i: (i, 0)),
        ],
        out_specs=pl.BlockSpec((1, 1, r), lambda i: (i, 0, 0)),
        out_shape=jax.ShapeDtypeStruct((nb, 1, r), jnp.float32),
    )(targets.reshape(nb, 1, r), log_probs)

    mesh = plsc.VectorSubcoreMesh(core_axis_name="c", subcore_axis_name="s",
                                  num_cores=1)
    reduce = pl.kernel(
        _sc_reduce_body, mesh=mesh,
        out_type=jax.ShapeDtypeStruct((_LANES,), jnp.float32),
        scratch_types=[
            pltpu.VMEM((_N,), jnp.float32),
            pltpu.VMEM((2 * _LANES,), jnp.float32),
            pltpu.VMEM((_LANES,), jnp.float32),
        ],
    )
    loss16 = reduce(contrib.reshape(n))
    return loss16[0]
